# Initial kernel scaffold; baseline (speedup 1.0000x reference)
import jax
import jax.numpy as jnp
from jax.experimental import pallas as pl


def kernel(input, adj_indices, adj_values, W_diag1, W_diag2, W1, b1, W2, b2):
    raise NotImplementedError("write your pallas kernel here")


# speedup vs baseline: 1.4398x; 1.4398x over previous
import functools
import jax
import jax.numpy as jnp
from jax.experimental import pallas as pl

_N = 10000
_F = 128
_K = 16
_NP = 10240
_BR = 256

_NEG = -3.0e38


def _topk_body(a_ref, b_ref, vals_ref, idx_ref):
    a = a_ref[...]
    b = b_ref[...]
    s = jnp.dot(a[:, :64], b[:64, :], preferred_element_type=jnp.float32)
    s = s + jnp.dot(a[:, 64:], b[64:, :], preferred_element_type=jnp.float32)
    col = jax.lax.broadcasted_iota(jnp.int32, s.shape, 1)
    s = jnp.where(col < _N, s, _NEG)
    for k in range(_K):
        m = jnp.max(s, axis=1, keepdims=True)
        ik = jnp.min(jnp.where(s == m, col, jnp.int32(2**30)), axis=1, keepdims=True)
        vals_ref[:, k : k + 1] = m
        idx_ref[:, k : k + 1] = ik
        s = jnp.where(col == ik, _NEG, s)


def _fused_topk(emb):
    emb_p = jnp.zeros((_NP, _F), jnp.float32).at[:_N].set(emb)
    emb_t = emb_p.T
    grid = (_NP // _BR,)
    vals, idx = pl.pallas_call(
        _topk_body,
        grid=grid,
        in_specs=[
            pl.BlockSpec((_BR, _F), lambda i: (i, 0)),
            pl.BlockSpec((_F, _NP), lambda i: (0, 0)),
        ],
        out_specs=[
            pl.BlockSpec((_BR, 128), lambda i: (i, 0)),
            pl.BlockSpec((_BR, 128), lambda i: (i, 0)),
        ],
        out_shape=[
            jax.ShapeDtypeStruct((_NP, 128), jnp.float32),
            jax.ShapeDtypeStruct((_NP, 128), jnp.int32),
        ],
    )(emb_p, emb_t)
    return vals[:_N, :_K], idx[:_N, :_K]


def _spmm(indices, values, x):
    gathered = jnp.take(x, indices[1], axis=0) * values[:, None]
    return jax.ops.segment_sum(gathered, indices[0], num_segments=_N)


def _normalize_adj(indices, values):
    deg = jax.ops.segment_sum(values, indices[0], num_segments=_N)
    inv_sqrt = 1.0 / (jnp.sqrt(deg) + 1e-10)
    return values * inv_sqrt[indices[0]] * inv_sqrt[indices[1]]


@jax.jit
def kernel(input, adj_indices, adj_values, W_diag1, W_diag2, W1, b1, W2, b2):
    norm_vals = _normalize_adj(adj_indices, adj_values)
    h = jnp.tanh(_spmm(adj_indices, norm_vals, input * W_diag1))
    emb = _spmm(adj_indices, norm_vals, h * W_diag2)
    nrm = jnp.linalg.norm(emb, axis=1, keepdims=True)
    emb = emb / jnp.maximum(nrm, 1e-12)

    vals, idx = _fused_topk(emb)

    rows = jnp.repeat(jnp.arange(_N, dtype=jnp.int32), _K)
    inds = jnp.stack([rows, idx.reshape(-1).astype(jnp.int32)])
    inds_sym = jnp.concatenate([inds, jnp.stack([inds[1], inds[0]])], axis=1)
    vals_flat = vals.reshape(-1)
    vals_sym = jnp.concatenate([vals_flat, vals_flat])

    new_inds = jnp.concatenate([adj_indices.astype(jnp.int32), inds_sym], axis=1)
    new_vals = jnp.concatenate([adj_values, vals_sym])
    norm_new = _normalize_adj(new_inds, new_vals)

    h1 = jax.nn.relu(_spmm(new_inds, norm_new, input @ W1 + b1))
    x_out = _spmm(new_inds, norm_new, h1 @ W2 + b2)
    return (x_out, inds_sym, vals_sym, new_inds, new_vals)


# speedup vs baseline: 4.5651x; 3.1707x over previous
import functools
import jax
import jax.numpy as jnp
from jax import lax
from jax.experimental import pallas as pl
from jax.experimental.pallas import tpu as pltpu, tpu_sc as plsc

_N = 10000
_F = 128
_K = 16
_NPAD = 10240
_BR = 256
_NEG = -3.0e38

_NC, _NS = 2, 16
_NW = _NC * _NS
_CH = 128
_DCH = 1024

@functools.lru_cache(maxsize=1)
def _mesh():
    return plsc.VectorSubcoreMesh(
        core_axis_name="c", subcore_axis_name="s", num_cores=_NC, num_subcores=_NS
    )


def _topk_body(a_ref, b_ref, vals_ref, idx_ref):
    a = a_ref[...]
    b = b_ref[...]
    s = jnp.dot(a[:, :64], b[:64, :], preferred_element_type=jnp.float32)
    s = s + jnp.dot(a[:, 64:], b[64:, :], preferred_element_type=jnp.float32)
    col = lax.broadcasted_iota(jnp.int32, s.shape, 1)
    s = jnp.where(col < _N, s, _NEG)
    for k in range(_K):
        m = jnp.max(s, axis=1, keepdims=True)
        ik = jnp.min(jnp.where(s == m, col, jnp.int32(2**30)), axis=1, keepdims=True)
        vals_ref[:, k : k + 1] = m
        idx_ref[:, k : k + 1] = ik
        s = jnp.where(col == ik, _NEG, s)


def _fused_topk(emb_pad):
    emb_t = emb_pad.T
    vals, idx = pl.pallas_call(
        _topk_body,
        grid=(_NPAD // _BR,),
        in_specs=[
            pl.BlockSpec((_BR, _F), lambda i: (i, 0)),
            pl.BlockSpec((_F, _NPAD), lambda i: (0, 0)),
        ],
        out_specs=[
            pl.BlockSpec((_BR, 128), lambda i: (i, 0)),
            pl.BlockSpec((_BR, 128), lambda i: (i, 0)),
        ],
        out_shape=[
            jax.ShapeDtypeStruct((_NPAD, 128), jnp.float32),
            jax.ShapeDtypeStruct((_NPAD, 128), jnp.int32),
        ],
    )(emb_pad, emb_t)
    return vals[:_N, :_K], idx[:_N, :_K]


def _deg_sc(i0p, valsp):
    ones_x = jnp.ones((_NPAD, 16), jnp.float32)
    ones_inv = jnp.ones((_NPAD,), jnp.float32)
    return _spmm_sc(ones_x, i0p, i0p, valsp, ones_inv)[:, 0]


def _spmm_body(d, nch, x_hbm, i0_hbm, i1_hbm, vals_hbm, inv_hbm, out_hbm,
               i0_v, i1_v, vals_v, svals_v, rows_v, inv_v, acc_sh, sem):
    c = lax.axis_index("c")
    s = lax.axis_index("s")
    wid = s * _NC + c
    rpt = _NPAD // _NS

    pltpu.sync_copy(inv_hbm, inv_v)

    def zrow(i, _):
        for j in range(d // 16):
            rows_v[i, pl.ds(j * 16, 16)] = jnp.zeros((16,), jnp.float32)
        return 0

    lax.fori_loop(0, _CH, zrow, 0)
    for r in range(rpt // _CH):
        pltpu.sync_copy(rows_v, acc_sh.at[pl.ds(s * rpt + r * _CH, _CH), :])
    plsc.subcore_barrier()

    base0 = wid * nch * _CH

    def chunk(k, _):
        base = base0 + k * _CH
        pltpu.sync_copy(i1_hbm.at[pl.ds(base, _CH)], i1_v)
        pltpu.async_copy(x_hbm.at[i1_v], rows_v, sem).wait()
        pltpu.sync_copy(i0_hbm.at[pl.ds(base, _CH)], i0_v)
        pltpu.sync_copy(vals_hbm.at[pl.ds(base, _CH)], vals_v)
        for g in range(_CH // 16):
            idx0 = i0_v[pl.ds(g * 16, 16)]
            idx1 = i1_v[pl.ds(g * 16, 16)]
            sv = (vals_v[pl.ds(g * 16, 16)]
                  * plsc.load_gather(inv_v, [idx0])
                  * plsc.load_gather(inv_v, [idx1]))
            svals_v[pl.ds(g * 16, 16)] = sv

        def rowscale(e, _):
            sv = plsc.load_gather(svals_v, [jnp.full((16,), e, jnp.int32)])
            for j in range(d // 16):
                rows_v[e, pl.ds(j * 16, 16)] = rows_v[e, pl.ds(j * 16, 16)] * sv
            return 0

        lax.fori_loop(0, _CH, rowscale, 0)
        pltpu.sync_copy(rows_v, acc_sh.at[i0_v], add=True)
        return 0

    lax.fori_loop(0, nch, chunk, 0)
    plsc.subcore_barrier()
    for r in range(rpt // _CH):
        sl = pl.ds(s * rpt + r * _CH, _CH)
        pltpu.sync_copy(acc_sh.at[sl, :], out_hbm.at[c, sl, :])


def _spmm_sc(x_pad, i0p, i1p, valsp, inv_pad):
    d = x_pad.shape[1]
    ep = i0p.shape[0]
    nch = ep // (_NW * _CH)
    parts = pl.kernel(
        functools.partial(_spmm_body, d, nch),
        out_type=jax.ShapeDtypeStruct((_NC, _NPAD, d), jnp.float32),
        mesh=_mesh(),
        compiler_params=pltpu.CompilerParams(
            needs_layout_passes=False, use_tc_tiling_on_sc=False
        ),
        scratch_types=[
            pltpu.VMEM((_CH,), jnp.int32),
            pltpu.VMEM((_CH,), jnp.int32),
            pltpu.VMEM((_CH,), jnp.float32),
            pltpu.VMEM((_CH,), jnp.float32),
            pltpu.VMEM((_CH, d), jnp.float32),
            pltpu.VMEM((_NPAD,), jnp.float32),
            pltpu.VMEM_SHARED((_NPAD, d), jnp.float32),
            pltpu.SemaphoreType.DMA,
        ],
    )(x_pad, i0p, i1p, valsp, inv_pad)
    return parts[0] + parts[1]


def _pad_edges(i0, i1, vals, ep):
    e = i0.shape[0]
    pad = ep - e
    i0p = jnp.concatenate([i0, jnp.full((pad,), _NPAD - 1, jnp.int32)])
    i1p = jnp.concatenate([i1, jnp.full((pad,), _NPAD - 1, jnp.int32)])
    valsp = jnp.concatenate([vals, jnp.zeros((pad,), jnp.float32)])
    return i0p, i1p, valsp


def _pad_rows(x):
    return jnp.zeros((_NPAD, x.shape[1]), jnp.float32).at[:_N].set(x)


@jax.jit
def kernel(input, adj_indices, adj_values, W_diag1, W_diag2, W1, b1, W2, b2):
    ep1 = 163840
    ep2 = 491520
    i0a, i1a, valsa = _pad_edges(adj_indices[0].astype(jnp.int32),
                                 adj_indices[1].astype(jnp.int32), adj_values, ep1)

    deg = _deg_sc(i0a, valsa)
    inv1 = 1.0 / (jnp.sqrt(deg) + 1e-10)

    x1 = _pad_rows(input * W_diag1)
    h = jnp.tanh(_spmm_sc(x1, i0a, i1a, valsa, inv1))
    emb = _spmm_sc(h * W_diag2, i0a, i1a, valsa, inv1)
    nrm = jnp.sqrt(jnp.sum(emb * emb, axis=1, keepdims=True))
    emb = emb / jnp.maximum(nrm, 1e-12)

    vals, idx = _fused_topk(emb)

    rows = jnp.repeat(jnp.arange(_N, dtype=jnp.int32), _K)
    inds = jnp.stack([rows, idx.reshape(-1).astype(jnp.int32)])
    inds_sym = jnp.concatenate([inds, jnp.stack([inds[1], inds[0]])], axis=1)
    vals_flat = vals.reshape(-1)
    vals_sym = jnp.concatenate([vals_flat, vals_flat])

    new_inds = jnp.concatenate([adj_indices.astype(jnp.int32), inds_sym], axis=1)
    new_vals = jnp.concatenate([adj_values, vals_sym])

    i0n, i1n, valsn = _pad_edges(new_inds[0], new_inds[1], new_vals, ep2)
    deg2 = _deg_sc(i0n, valsn)
    inv2 = 1.0 / (jnp.sqrt(deg2) + 1e-10)

    xw1 = _pad_rows(input @ W1 + b1)
    h1 = jax.nn.relu(_spmm_sc(xw1, i0n, i1n, valsn, inv2))
    h1w2 = h1 @ W2 + b2
    x_out = _spmm_sc(h1w2, i0n, i1n, valsn, inv2)[:_N]

    return (x_out, inds_sym, vals_sym, new_inds, new_vals)


# speedup vs baseline: 6.1871x; 1.3553x over previous
import functools
import jax
import jax.numpy as jnp
from jax import lax
from jax.experimental import pallas as pl
from jax.experimental.pallas import tpu as pltpu, tpu_sc as plsc

_N = 10000
_F = 128
_K = 16
_NPAD = 10240
_BR = 256
_NEG = -3.0e38

_NC, _NS = 2, 16
_NW = _NC * _NS
_CH = 128
_DCH = 1024

@functools.lru_cache(maxsize=1)
def _mesh():
    return plsc.VectorSubcoreMesh(
        core_axis_name="c", subcore_axis_name="s", num_cores=_NC, num_subcores=_NS
    )


def _topk_body(a_ref, b_ref, vals_ref, idx_ref):
    a = a_ref[...]
    b = b_ref[...]
    s = jnp.dot(a[:, :64], b[:64, :], preferred_element_type=jnp.float32)
    s = s + jnp.dot(a[:, 64:], b[64:, :], preferred_element_type=jnp.float32)
    col = lax.broadcasted_iota(jnp.int32, s.shape, 1)
    s = jnp.where(col < _N, s, _NEG)
    for k in range(_K):
        m = jnp.max(s, axis=1, keepdims=True)
        ik = jnp.min(jnp.where(s == m, col, jnp.int32(2**30)), axis=1, keepdims=True)
        vals_ref[:, k : k + 1] = m
        idx_ref[:, k : k + 1] = ik
        s = jnp.where(col == ik, _NEG, s)


def _fused_topk(emb_pad):
    emb_t = emb_pad.T
    vals, idx = pl.pallas_call(
        _topk_body,
        grid=(_NPAD // _BR,),
        in_specs=[
            pl.BlockSpec((_BR, _F), lambda i: (i, 0)),
            pl.BlockSpec((_F, _NPAD), lambda i: (0, 0)),
        ],
        out_specs=[
            pl.BlockSpec((_BR, 128), lambda i: (i, 0)),
            pl.BlockSpec((_BR, 128), lambda i: (i, 0)),
        ],
        out_shape=[
            jax.ShapeDtypeStruct((_NPAD, 128), jnp.float32),
            jax.ShapeDtypeStruct((_NPAD, 128), jnp.int32),
        ],
    )(emb_pad, emb_t)
    return vals[:_N, :_K], idx[:_N, :_K]


def _deg_body(nsup, i02_hbm, vals_hbm, out_hbm,
              i0S, valsS, rows0, rows1, acc_sh, sem_i, sem_s0, sem_s1):
    c = lax.axis_index("c")
    s = lax.axis_index("s")
    wid = s * _NC + c

    _zero_acc(rows0, acc_sh, s, 16)
    plsc.subcore_barrier()

    rows = (rows0, rows1)
    sem_s = (sem_s0, sem_s1)
    base0 = wid * nsup

    def super_chunk(S, _):
        row0 = (base0 + S) * _SCH
        base = (base0 + S) * _SUP
        di0 = pltpu.async_copy(i02_hbm.at[pl.ds(row0, _SCH), :], i0S, sem_i)
        dv = pltpu.async_copy(vals_hbm.at[pl.ds(base, _SUP)], valsS, sem_i)
        di0.wait()
        dv.wait()
        sct = [None, None]
        for j in range(_SCH):
            p = j % 2
            if sct[p] is not None:
                sct[p].wait()
            rv = rows[p]

            def rowfill(e, _):
                v = plsc.load_gather(
                    valsS, [jnp.full((16,), j * _CH + e, jnp.int32)])
                rv[e, pl.ds(0, 16)] = v
                return 0

            lax.fori_loop(0, _CH, rowfill, 0)
            sct[p] = pltpu.async_copy(
                rows[p], acc_sh.at[i0S.at[j]], sem_s[p], add=True)
        sct[0].wait()
        sct[1].wait()
        return 0

    lax.fori_loop(0, nsup, super_chunk, 0)
    plsc.subcore_barrier()
    _write_out(acc_sh, out_hbm, c, s)


def _deg_sc(i0p, valsp):
    ep = i0p.shape[0]
    nsup = ep // (_NW * _SUP)
    parts = pl.kernel(
        functools.partial(_deg_body, nsup),
        out_type=jax.ShapeDtypeStruct((_NC, _NPAD, 16), jnp.float32),
        mesh=_mesh(),
        compiler_params=pltpu.CompilerParams(
            needs_layout_passes=False, use_tc_tiling_on_sc=False
        ),
        scratch_types=[
            pltpu.VMEM((_SCH, _CH), jnp.int32),
            pltpu.VMEM((_SUP,), jnp.float32),
            pltpu.VMEM((_CH, 16), jnp.float32),
            pltpu.VMEM((_CH, 16), jnp.float32),
            pltpu.VMEM_SHARED((_NPAD, 16), jnp.float32),
            pltpu.SemaphoreType.DMA,
            pltpu.SemaphoreType.DMA,
            pltpu.SemaphoreType.DMA,
        ],
    )(i0p.reshape(ep // _CH, _CH), valsp)
    return (parts[0] + parts[1])[:, 0]


_SCH = 8
_SUP = _CH * _SCH


def _zero_acc(rows0, acc_sh, s, d):
    rpt = _NPAD // _NS

    def zrow(i, _):
        for j in range(d // 16):
            rows0[i, pl.ds(j * 16, 16)] = jnp.zeros((16,), jnp.float32)
        return 0

    lax.fori_loop(0, _CH, zrow, 0)
    for r in range(rpt // _CH):
        pltpu.sync_copy(rows0, acc_sh.at[pl.ds(s * rpt + r * _CH, _CH), :])


def _write_out(acc_sh, out_hbm, c, s):
    rpt = _NPAD // _NS
    for r in range(rpt // _CH):
        sl = pl.ds(s * rpt + r * _CH, _CH)
        pltpu.sync_copy(acc_sh.at[sl, :], out_hbm.at[c, sl, :])


def _spmm_body(d, nsup, x_hbm, i02_hbm, i1_hbm, vals_hbm, inv_hbm, out_hbm,
               i0S, i1S, valsS, svalsC, rows0, rows1, inv_v, acc_sh,
               sem_i, sem_g0, sem_g1, sem_s0, sem_s1):
    c = lax.axis_index("c")
    s = lax.axis_index("s")
    wid = s * _NC + c

    pltpu.sync_copy(inv_hbm, inv_v)
    _zero_acc(rows0, acc_sh, s, d)
    plsc.subcore_barrier()

    rows = (rows0, rows1)
    sem_g = (sem_g0, sem_g1)
    sem_s = (sem_s0, sem_s1)
    base0 = wid * nsup

    def super_chunk(S, _):
        row0 = (base0 + S) * _SCH
        base = (base0 + S) * _SUP
        di0 = pltpu.async_copy(i02_hbm.at[pl.ds(row0, _SCH), :], i0S, sem_i)
        di1 = pltpu.async_copy(i1_hbm.at[pl.ds(base, _SUP)], i1S, sem_i)
        dv = pltpu.async_copy(vals_hbm.at[pl.ds(base, _SUP)], valsS, sem_i)
        di0.wait()
        di1.wait()
        dv.wait()

        g = [None, None]
        sct = [None, None]
        g[0] = pltpu.async_copy(x_hbm.at[i1S.at[pl.ds(0, _CH)]], rows0, sem_g0)
        for j in range(_SCH):
            p = j % 2
            if j < _SCH - 1:
                q = (j + 1) % 2
                if sct[q] is not None:
                    sct[q].wait()
                g[q] = pltpu.async_copy(
                    x_hbm.at[i1S.at[pl.ds((j + 1) * _CH, _CH)]], rows[q],
                    sem_g[q])
            g[p].wait()
            for grp in range(_CH // 16):
                idx0 = i0S[j, pl.ds(grp * 16, 16)]
                idx1 = i1S[pl.ds(j * _CH + grp * 16, 16)]
                sv = (valsS[pl.ds(j * _CH + grp * 16, 16)]
                      * plsc.load_gather(inv_v, [idx0])
                      * plsc.load_gather(inv_v, [idx1]))
                svalsC[pl.ds(grp * 16, 16)] = sv

            rv = rows[p]

            def rowscale(e, _):
                sv = plsc.load_gather(svalsC, [jnp.full((16,), e, jnp.int32)])
                for jj in range(d // 16):
                    rv[e, pl.ds(jj * 16, 16)] = rv[e, pl.ds(jj * 16, 16)] * sv
                return 0

            lax.fori_loop(0, _CH, rowscale, 0)
            sct[p] = pltpu.async_copy(
                rows[p], acc_sh.at[i0S.at[j]], sem_s[p], add=True)
        sct[0].wait()
        sct[1].wait()
        return 0

    lax.fori_loop(0, nsup, super_chunk, 0)
    plsc.subcore_barrier()
    _write_out(acc_sh, out_hbm, c, s)


def _spmm_sc(x_pad, i0p, i1p, valsp, inv_pad):
    d = x_pad.shape[1]
    ep = i0p.shape[0]
    nsup = ep // (_NW * _SUP)
    parts = pl.kernel(
        functools.partial(_spmm_body, d, nsup),
        out_type=jax.ShapeDtypeStruct((_NC, _NPAD, d), jnp.float32),
        mesh=_mesh(),
        compiler_params=pltpu.CompilerParams(
            needs_layout_passes=False, use_tc_tiling_on_sc=False
        ),
        scratch_types=[
            pltpu.VMEM((_SCH, _CH), jnp.int32),
            pltpu.VMEM((_SUP,), jnp.int32),
            pltpu.VMEM((_SUP,), jnp.float32),
            pltpu.VMEM((_CH,), jnp.float32),
            pltpu.VMEM((_CH, d), jnp.float32),
            pltpu.VMEM((_CH, d), jnp.float32),
            pltpu.VMEM((_NPAD,), jnp.float32),
            pltpu.VMEM_SHARED((_NPAD, d), jnp.float32),
            pltpu.SemaphoreType.DMA,
            pltpu.SemaphoreType.DMA,
            pltpu.SemaphoreType.DMA,
            pltpu.SemaphoreType.DMA,
            pltpu.SemaphoreType.DMA,
        ],
    )(x_pad, i0p.reshape(ep // _CH, _CH), i1p, valsp, inv_pad)
    return parts[0] + parts[1]


def _pad_edges(i0, i1, vals, ep):
    e = i0.shape[0]
    pad = ep - e
    i0p = jnp.concatenate([i0, jnp.full((pad,), _NPAD - 1, jnp.int32)])
    i1p = jnp.concatenate([i1, jnp.full((pad,), _NPAD - 1, jnp.int32)])
    valsp = jnp.concatenate([vals, jnp.zeros((pad,), jnp.float32)])
    return i0p, i1p, valsp


def _pad_rows(x):
    return jnp.zeros((_NPAD, x.shape[1]), jnp.float32).at[:_N].set(x)


@jax.jit
def kernel(input, adj_indices, adj_values, W_diag1, W_diag2, W1, b1, W2, b2):
    ep1 = 163840
    ep2 = 491520
    i0a, i1a, valsa = _pad_edges(adj_indices[0].astype(jnp.int32),
                                 adj_indices[1].astype(jnp.int32), adj_values, ep1)

    deg = _deg_sc(i0a, valsa)
    inv1 = 1.0 / (jnp.sqrt(deg) + 1e-10)

    x1 = _pad_rows(input * W_diag1)
    h = jnp.tanh(_spmm_sc(x1, i0a, i1a, valsa, inv1))
    emb = _spmm_sc(h * W_diag2, i0a, i1a, valsa, inv1)
    nrm = jnp.sqrt(jnp.sum(emb * emb, axis=1, keepdims=True))
    emb = emb / jnp.maximum(nrm, 1e-12)

    vals, idx = _fused_topk(emb)

    rows = jnp.repeat(jnp.arange(_N, dtype=jnp.int32), _K)
    inds = jnp.stack([rows, idx.reshape(-1).astype(jnp.int32)])
    inds_sym = jnp.concatenate([inds, jnp.stack([inds[1], inds[0]])], axis=1)
    vals_flat = vals.reshape(-1)
    vals_sym = jnp.concatenate([vals_flat, vals_flat])

    new_inds = jnp.concatenate([adj_indices.astype(jnp.int32), inds_sym], axis=1)
    new_vals = jnp.concatenate([adj_values, vals_sym])

    i0n, i1n, valsn = _pad_edges(new_inds[0], new_inds[1], new_vals, ep2)
    deg2 = _deg_sc(i0n, valsn)
    inv2 = 1.0 / (jnp.sqrt(deg2) + 1e-10)

    xw1 = _pad_rows(input @ W1 + b1)
    h1 = jax.nn.relu(_spmm_sc(xw1, i0n, i1n, valsn, inv2))
    h1w2 = h1 @ W2 + b2
    x_out = _spmm_sc(h1w2, i0n, i1n, valsn, inv2)[:_N]

    return (x_out, inds_sym, vals_sym, new_inds, new_vals)
